# Initial kernel scaffold; baseline (speedup 1.0000x reference)
import functools

import jax
import jax.numpy as jnp
from jax import lax
from jax.experimental import pallas as pl
from jax.experimental.pallas import tpu as pltpu
from jax.experimental.pallas import tpu_sc as plsc

N = 100000
E = 3200000
NUM_FEAT = 16
DIM = 8
H1 = 8

ET = E + N
NC, NS = 2, 16
NW = NC * NS
B = 1024
CH = B // 128
NBLK = -(-ET // (NW * B))
ETP = NW * B * NBLK
PER_W = ETP // NW
PAD = ETP - ET
NP1 = 100032
ROWS_PT = NP1 // NS
RB = 1000
GRID = N // RB

_mesh = plsc.VectorSubcoreMesh(
    core_axis_name="c", subcore_axis_name="s", num_cores=NC, num_subcores=NS
)
_PIB = lax.GatherScatterMode.PROMISE_IN_BOUNDS

f32 = jnp.float32
i32 = jnp.int32


def _leaky(x):
  return jnp.where(x >= 0, x, 0.2 * x)


def _tc1_body(x_ref, w1_ref, pas_ref, pad_ref, h_ref, t1_ref, bm_ref):
  h = jnp.dot(x_ref[...], w1_ref[...], preferred_element_type=f32)
  h_ref[...] = h
  a_s = jnp.dot(h, pas_ref[...], preferred_element_type=f32)
  a_d = jnp.dot(h, pad_ref[...], preferred_element_type=f32)
  t1_ref[...] = jnp.concatenate([a_s, a_d], axis=1)
  bm_ref[...] = jnp.concatenate(
      [jnp.full((1, 64), jnp.max(a_s), f32),
       jnp.full((1, 64), jnp.max(a_d), f32)], axis=1)


def _tc1(x, w1, pas, padm):
  return pl.pallas_call(
      _tc1_body,
      grid=(GRID,),
      in_specs=[
          pl.BlockSpec((RB, NUM_FEAT), lambda i: (i, 0)),
          pl.BlockSpec((NUM_FEAT, 64), lambda i: (0, 0)),
          pl.BlockSpec((64, 8), lambda i: (0, 0)),
          pl.BlockSpec((64, 8), lambda i: (0, 0)),
      ],
      out_specs=[
          pl.BlockSpec((RB, 64), lambda i: (i, 0)),
          pl.BlockSpec((RB, 16), lambda i: (i, 0)),
          pl.BlockSpec((1, 128), lambda i: (i, 0)),
      ],
      out_shape=[
          jax.ShapeDtypeStruct((N, 64), f32),
          jax.ShapeDtypeStruct((N, 16), f32),
          jax.ShapeDtypeStruct((GRID, 128), f32),
      ],
  )(x, w1, pas, padm)


@functools.partial(
    pl.kernel,
    out_type=(
        jax.ShapeDtypeStruct((4, ETP, 2), f32),
        jax.ShapeDtypeStruct((NC, NP1, 8), f32),
    ),
    mesh=_mesh,
    scratch_types=[
        pltpu.VMEM((CH, 128), i32),
        pltpu.VMEM((CH, 128), i32),
        pltpu.VMEM((B, 16), f32),
        pltpu.VMEM((B, 16), f32),
        pltpu.VMEM((B, 8), f32),
        pltpu.VMEM((4, B, 2), f32),
        pltpu.VMEM((16,), f32),
        pltpu.VMEM_SHARED((NP1, 8), f32),
    ],
)
def _pass_a(t1_hbm, src_hbm, dst_hbm, m_hbm, z8_hbm,
            wt_hbm, den_hbm,
            sidx, didx, gs, gd, wrows, wt, mvec, denacc):
  cid = lax.axis_index("c")
  sid = lax.axis_index("s")
  wid = sid * NC + cid
  pltpu.sync_copy(z8_hbm, denacc.at[pl.ds(sid * ROWS_PT, ROWS_PT)])
  pltpu.sync_copy(m_hbm, mvec)
  plsc.subcore_barrier()
  mv = mvec[...]
  iota = lax.iota(i32, 16)
  rot8 = iota ^ 8
  lt8 = iota < 8
  ibase = ((iota & 7) >> 1) * (2 * B) + 2 * (iota >> 3) + (iota & 1)
  gsf = gs.reshape(B * 16)
  gdf = gd.reshape(B * 16)
  wrf = wrows.reshape(B * 8)
  wtf = wt.reshape(8 * B)
  wrow_base = wid * (PER_W // 128)

  @pl.loop(0, NBLK)
  def _blk(blk):
    rowb = wrow_base + blk * CH
    pltpu.sync_copy(src_hbm.at[pl.ds(rowb, CH)], sidx)
    pltpu.sync_copy(dst_hbm.at[pl.ds(rowb, CH)], didx)
    for c in range(CH):
      pltpu.sync_copy(t1_hbm.at[sidx.at[c]], gs.at[pl.ds(c * 128, 128)])
      pltpu.sync_copy(t1_hbm.at[didx.at[c]], gd.at[pl.ds(c * 128, 128)])

    @pl.loop(0, B // 2)
    def _pair(i):
      v0 = gsf[pl.ds(i * 32, 16)]
      v1 = gsf[pl.ds(i * 32 + 16, 16)]
      u0 = gdf[pl.ds(i * 32, 16)]
      u1 = gdf[pl.ds(i * 32 + 16, 16)]
      asv = jnp.where(lt8, v0, jnp.take(v1, rot8, mode=_PIB))
      adv = jnp.where(lt8, jnp.take(u0, rot8, mode=_PIB), u1)
      w = jnp.exp(_leaky(asv + adv) - mv)
      wrf[pl.ds(i * 16, 16)] = w
      plsc.store_scatter(wtf, [ibase + i * 4], w)

    for c in range(CH):
      pltpu.sync_copy(wrows.at[pl.ds(c * 128, 128)],
                      denacc.at[didx.at[c]], add=True)
    ebase = wid * PER_W + blk * B
    for hp in range(4):
      pltpu.sync_copy(wt.at[hp], wt_hbm.at[hp, pl.ds(ebase, B)])

  plsc.subcore_barrier()
  pltpu.sync_copy(denacc.at[pl.ds(sid * ROWS_PT, ROWS_PT)],
                  den_hbm.at[cid, pl.ds(sid * ROWS_PT, ROWS_PT)])


def _make_pass_b(hp):
  @functools.partial(
      pl.kernel,
      out_type=jax.ShapeDtypeStruct((NC, NP1, 16), f32),
      mesh=_mesh,
      scratch_types=[
          pltpu.VMEM((CH, 128), i32),
          pltpu.VMEM((CH, 128), i32),
          pltpu.VMEM((CH, 128), i32),
          pltpu.VMEM((B, 16), f32),
          pltpu.VMEM((B, 2), f32),
          pltpu.VMEM((B, 16), f32),
          pltpu.VMEM_SHARED((NP1, 16), f32),
      ],
  )
  def _pass_b(ht_hbm, src_hbm, dst_hbm, w_hbm, z16_hbm, out_hbm,
              sidx, didx, idx4, gh, wv, msg, acc):
    cid = lax.axis_index("c")
    sid = lax.axis_index("s")
    wid = sid * NC + cid
    pltpu.sync_copy(z16_hbm, acc.at[pl.ds(sid * ROWS_PT, ROWS_PT)])
    plsc.subcore_barrier()
    iota = lax.iota(i32, 16)
    basepat = iota >> 3
    sidxf = sidx.reshape(CH * 128)
    idx4f = idx4.reshape(CH * 128)
    ghf = gh.reshape(B * 16)
    wvf = wv.reshape(B * 2)
    msgf = msg.reshape(B * 16)
    wrow_base = wid * (PER_W // 128)

    @pl.loop(0, NBLK)
    def _blk(blk):
      rowb = wrow_base + blk * CH
      pltpu.sync_copy(src_hbm.at[pl.ds(rowb, CH)], sidx)
      pltpu.sync_copy(dst_hbm.at[pl.ds(rowb, CH)], didx)

      @pl.loop(0, B // 16)
      def _cvt(j):
        idx4f[pl.ds(j * 16, 16)] = sidxf[pl.ds(j * 16, 16)] * 4 + hp

      for c in range(CH):
        pltpu.sync_copy(ht_hbm.at[idx4.at[c]], gh.at[pl.ds(c * 128, 128)])
      ebase = wid * PER_W + blk * B
      pltpu.sync_copy(w_hbm.at[hp, pl.ds(ebase, B)], wv)

      @pl.loop(0, B // 8)
      def _grp(g):
        wvec = wvf[pl.ds(g * 16, 16)]
        for t in range(8):
          m = jnp.take(wvec, basepat + 2 * t, mode=_PIB)
          gv = ghf[pl.ds((g * 8 + t) * 16, 16)]
          msgf[pl.ds((g * 8 + t) * 16, 16)] = gv * m

      for c in range(CH):
        pltpu.sync_copy(msg.at[pl.ds(c * 128, 128)],
                        acc.at[didx.at[c]], add=True)

    plsc.subcore_barrier()
    pltpu.sync_copy(acc.at[pl.ds(sid * ROWS_PT, ROWS_PT)],
                    out_hbm.at[cid, pl.ds(sid * ROWS_PT, ROWS_PT)])

  return _pass_b


_pass_b_fns = [_make_pass_b(hp) for hp in range(4)]


def _fin1_body(da_ref, db_ref, p00, p01, p02, p03, p10, p11, p12, p13,
               rep_ref, w2_ref, as2_ref, ad2_ref, b1_ref,
               t2_ref, bm2_ref):
  den = da_ref[...] + db_ref[...] + 1e-16
  msum = jnp.concatenate(
      [p00[...] + p10[...], p01[...] + p11[...],
       p02[...] + p12[...], p03[...] + p13[...]], axis=1)
  drep = jnp.dot(den, rep_ref[...], preferred_element_type=f32)
  out1 = msum / drep + b1_ref[...]
  hmid = jnp.where(out1 > 0, out1, jnp.expm1(out1))
  h2 = jnp.dot(hmid, w2_ref[...], preferred_element_type=f32)
  a2s = jnp.sum(h2 * as2_ref[...], axis=1, keepdims=True)
  a2d = jnp.sum(h2 * ad2_ref[...], axis=1, keepdims=True)
  t2_ref[...] = jnp.concatenate(
      [h2, a2s, a2d, jnp.zeros((RB, 6), f32)], axis=1)
  bm2_ref[...] = jnp.concatenate(
      [jnp.full((1, 64), jnp.max(a2s), f32),
       jnp.full((1, 64), jnp.max(a2d), f32)], axis=1)


def _fin1(den, parts, rep, w2, as2, ad2, b1row):
  def full(shape):
    return pl.BlockSpec(shape, lambda i: tuple(0 for _ in shape))
  return pl.pallas_call(
      _fin1_body,
      grid=(GRID,),
      in_specs=[
          pl.BlockSpec((RB, 8), lambda i: (i, 0)),
          pl.BlockSpec((RB, 8), lambda i: (i, 0)),
      ] + [pl.BlockSpec((RB, 16), lambda i: (i, 0))] * 8 + [
          full((8, 64)), full((64, 8)), full((1, 8)), full((1, 8)),
          full((1, 64)),
      ],
      out_specs=[
          pl.BlockSpec((RB, 16), lambda i: (i, 0)),
          pl.BlockSpec((1, 128), lambda i: (i, 0)),
      ],
      out_shape=[
          jax.ShapeDtypeStruct((N, 16), f32),
          jax.ShapeDtypeStruct((GRID, 128), f32),
      ],
  )(den[0], den[1], *parts, rep, w2, as2, ad2, b1row)


@functools.partial(
    pl.kernel,
    out_type=jax.ShapeDtypeStruct((NC, NP1, 16), f32),
    mesh=_mesh,
    scratch_types=[
        pltpu.VMEM((CH, 128), i32),
        pltpu.VMEM((CH, 128), i32),
        pltpu.VMEM((B, 16), f32),
        pltpu.VMEM((B, 16), f32),
        pltpu.VMEM((B, 16), f32),
        pltpu.VMEM((16,), f32),
        pltpu.VMEM_SHARED((NP1, 16), f32),
    ],
)
def _pass_c(t2_hbm, src_hbm, dst_hbm, m_hbm, z16_hbm, out_hbm,
            sidx, didx, gs, gd, msg, mvec, acc):
  cid = lax.axis_index("c")
  sid = lax.axis_index("s")
  wid = sid * NC + cid
  pltpu.sync_copy(z16_hbm, acc.at[pl.ds(sid * ROWS_PT, ROWS_PT)])
  pltpu.sync_copy(m_hbm, mvec)
  plsc.subcore_barrier()
  mv = mvec[...]
  iota = lax.iota(i32, 16)
  rot1 = (iota + 1) & 15
  pat8 = (iota & 0) + 8
  c0 = jnp.where(iota < 8, 1.0, 0.0).astype(f32)
  c1 = jnp.where(iota == 8, 1.0, 0.0).astype(f32)
  gsf = gs.reshape(B * 16)
  gdf = gd.reshape(B * 16)
  msgf = msg.reshape(B * 16)
  wrow_base = wid * (PER_W // 128)

  @pl.loop(0, NBLK)
  def _blk(blk):
    rowb = wrow_base + blk * CH
    pltpu.sync_copy(src_hbm.at[pl.ds(rowb, CH)], sidx)
    pltpu.sync_copy(dst_hbm.at[pl.ds(rowb, CH)], didx)
    for c in range(CH):
      pltpu.sync_copy(t2_hbm.at[sidx.at[c]], gs.at[pl.ds(c * 128, 128)])
      pltpu.sync_copy(t2_hbm.at[didx.at[c]], gd.at[pl.ds(c * 128, 128)])

    @pl.loop(0, B)
    def _edge(i):
      v_s = gsf[pl.ds(i * 16, 16)]
      v_d = gdf[pl.ds(i * 16, 16)]
      t = v_s + jnp.take(v_d, rot1, mode=_PIB)
      w = jnp.exp(_leaky(t) - mv)
      b = jnp.take(w, pat8, mode=_PIB)
      msgf[pl.ds(i * 16, 16)] = (v_s * c0 + c1) * b

    for c in range(CH):
      pltpu.sync_copy(msg.at[pl.ds(c * 128, 128)],
                      acc.at[didx.at[c]], add=True)

  plsc.subcore_barrier()
  pltpu.sync_copy(acc.at[pl.ds(sid * ROWS_PT, ROWS_PT)],
                  out_hbm.at[cid, pl.ds(sid * ROWS_PT, ROWS_PT)])


def _fin2_body(aa_ref, ab_ref, b2_ref, out_ref):
  s = aa_ref[...] + ab_ref[...]
  out_ref[...] = s[:, :8] / (s[:, 8:9] + 1e-16) + b2_ref[...]


def _fin2(acc2, b2row):
  return pl.pallas_call(
      _fin2_body,
      grid=(GRID,),
      in_specs=[
          pl.BlockSpec((RB, 16), lambda i: (i, 0)),
          pl.BlockSpec((RB, 16), lambda i: (i, 0)),
          pl.BlockSpec((1, 8), lambda i: (0, 0)),
      ],
      out_specs=pl.BlockSpec((RB, 8), lambda i: (i, 0)),
      out_shape=jax.ShapeDtypeStruct((N, 8), f32),
  )(acc2[0], acc2[1], b2row)


def kernel(x, edge_index, W1, a_src1, a_dst1, b1, W2, a_src2, a_dst2, b2):
  loops = jnp.arange(N, dtype=i32)
  src = jnp.concatenate([edge_index[0], loops, jnp.zeros((PAD,), i32)])
  dst = jnp.concatenate([edge_index[1], loops, jnp.full((PAD,), N, i32)])
  src2d = src.reshape(ETP // 128, 128)
  dst2d = dst.reshape(ETP // 128, 128)

  eye8 = jnp.eye(8, dtype=f32)
  pas = (eye8[:, None, :] * a_src1[:, :, None]).reshape(64, 8)
  padm = (eye8[:, None, :] * a_dst1[:, :, None]).reshape(64, 8)
  rep = jnp.repeat(eye8, 8, axis=1)

  z8 = jnp.zeros((ROWS_PT, 8), f32)
  z16 = jnp.zeros((ROWS_PT, 16), f32)

  h1, t1, bm1 = _tc1(x, W1, pas, padm)
  m1 = jnp.max(bm1[:, 0]) + jnp.max(bm1[:, 64])
  m1v = jnp.full((16,), m1, f32)
  t1p = jnp.concatenate([t1, jnp.zeros((1, 16), f32)], axis=0)
  ht = jnp.concatenate([h1, jnp.zeros((1, 64), f32)], axis=0)
  ht = ht.reshape((N + 1) * 4, 16)

  wt, den = _pass_a(t1p, src2d, dst2d, m1v, z8)
  bparts = [fn(ht, src2d, dst2d, wt, z16) for fn in _pass_b_fns]
  parts = [p[c][:N] for c in range(NC) for p in bparts]

  t2, bm2 = _fin1((den[0][:N], den[1][:N]), parts, rep, W2,
                  a_src2.reshape(1, 8), a_dst2.reshape(1, 8),
                  b1.reshape(1, 64))
  m2 = jnp.max(bm2[:, 0]) + jnp.max(bm2[:, 64])
  m2v = jnp.full((16,), m2, f32)
  t2p = jnp.concatenate([t2, jnp.zeros((1, 16), f32)], axis=0)

  acc2 = _pass_c(t2p, src2d, dst2d, m2v, z16)

  return _fin2((acc2[0][:N], acc2[1][:N]), b2.reshape(1, 8))


# speedup vs baseline: 58.2846x; 58.2846x over previous
import dataclasses
import functools

import jax
import jax.numpy as jnp
from jax import lax
from jax.experimental import pallas as pl
from jax.experimental.pallas import tpu as pltpu
from jax.experimental.pallas import tpu_sc as plsc

N = 100000
E = 3200000
NUM_FEAT = 16
DIM = 8
H1 = 8

ET = E + N
NC, NS = 2, 16
NW = NC * NS
B = 1024
CH = B // 128
NBLK = -(-ET // (NW * B))
ETP = NW * B * NBLK
B2 = 512
CH2 = B2 // 128
NBLK2 = ETP // (NW * B2)
PER_W = ETP // NW
PAD = ETP - ET
NP1 = 100096
ROWS_PT = NP1 // NS
RB = 1000
GRID = N // RB

_mesh = plsc.VectorSubcoreMesh(
    core_axis_name="c", subcore_axis_name="s", num_cores=NC, num_subcores=NS
)

f32 = jnp.float32
i32 = jnp.int32

_sc_params = pltpu.CompilerParams(use_tc_tiling_on_sc=False)
if "needs_layout_passes" in pltpu.CompilerParams.__dataclass_fields__:
  _sc_params = dataclasses.replace(_sc_params, needs_layout_passes=False)


def _leaky(x):
  return jnp.where(x >= 0, x, 0.2 * x)


def _shuf(v, idx):
  return jnp.take_along_axis(v, idx, axis=0, mode="promise_in_bounds")


def _tc1_body(x_ref, w1_ref, pas_ref, pad_ref, h_ref, t1_ref, bm_ref):
  h = jnp.dot(x_ref[...], w1_ref[...], preferred_element_type=f32)
  h_ref[...] = h
  a_s = jnp.dot(h, pas_ref[...], preferred_element_type=f32)
  a_d = jnp.dot(h, pad_ref[...], preferred_element_type=f32)
  t1_ref[...] = jnp.concatenate([a_s, a_d], axis=1)
  bm_ref[...] = jnp.concatenate(
      [jnp.full((1, 8, 64), jnp.max(a_s), f32),
       jnp.full((1, 8, 64), jnp.max(a_d), f32)], axis=2)


def _tc1(x, w1, pas, padm):
  return pl.pallas_call(
      _tc1_body,
      grid=(GRID,),
      in_specs=[
          pl.BlockSpec((RB, NUM_FEAT), lambda i: (i, 0)),
          pl.BlockSpec((NUM_FEAT, 64), lambda i: (0, 0)),
          pl.BlockSpec((64, 8), lambda i: (0, 0)),
          pl.BlockSpec((64, 8), lambda i: (0, 0)),
      ],
      out_specs=[
          pl.BlockSpec((RB, 64), lambda i: (i, 0)),
          pl.BlockSpec((RB, 16), lambda i: (i, 0)),
          pl.BlockSpec((1, 8, 128), lambda i: (i, 0, 0)),
      ],
      out_shape=[
          jax.ShapeDtypeStruct((N, 64), f32),
          jax.ShapeDtypeStruct((N, 16), f32),
          jax.ShapeDtypeStruct((GRID, 8, 128), f32),
      ],
  )(x, w1, pas, padm)


@functools.partial(
    pl.kernel,
    out_type=(
        jax.ShapeDtypeStruct((4, ETP // 8, 16), f32),
        jax.ShapeDtypeStruct((NC, NP1, 8), f32),
    ),
    mesh=_mesh,
    compiler_params=_sc_params,
    scratch_types=[
        pltpu.VMEM((CH, 128), i32),
        pltpu.VMEM((CH, 128), i32),
        pltpu.VMEM((B, 16), f32),
        pltpu.VMEM((B, 16), f32),
        pltpu.VMEM((B, 8), f32),
        pltpu.VMEM((4, B // 8, 16), f32),
        pltpu.VMEM((16,), f32),
        pltpu.VMEM_SHARED((NP1, 8), f32),
    ],
)
def _pass_a(t1_hbm, src_hbm, dst_hbm, m_hbm, z8_hbm,
            wt_hbm, den_hbm,
            sidx, didx, gs, gd, wrows, wt, mvec, denacc):
  cid = lax.axis_index("c")
  sid = lax.axis_index("s")
  wid = sid * NC + cid
  pltpu.sync_copy(z8_hbm, denacc.at[pl.ds(sid * ROWS_PT, ROWS_PT)])
  pltpu.sync_copy(m_hbm, mvec)
  plsc.subcore_barrier()
  mv = mvec[...]
  iota = lax.iota(i32, 16)
  rot8 = iota ^ 8
  lt8 = iota < 8
  colw = iota & 7
  half = iota >> 3
  hpvec = (iota & 7) >> 1
  qoff = 2 * half + (iota & 1)
  wrow_base = wid * (PER_W // 128)

  @pl.loop(0, NBLK)
  def _blk(blk):
    rowb = wrow_base + blk * CH
    pltpu.sync_copy(src_hbm.at[pl.ds(rowb, CH)], sidx)
    pltpu.sync_copy(dst_hbm.at[pl.ds(rowb, CH)], didx)
    for c in range(CH):
      pltpu.sync_copy(t1_hbm.at[sidx.at[c]], gs.at[pl.ds(c * 128, 128)])
      pltpu.sync_copy(t1_hbm.at[didx.at[c]], gd.at[pl.ds(c * 128, 128)])

    @pl.loop(0, B // 2)
    def _pair(i):
      v0 = gs[2 * i]
      v1 = gs[2 * i + 1]
      u0 = gd[2 * i]
      u1 = gd[2 * i + 1]
      asv = jnp.where(lt8, v0, _shuf(v1, rot8))
      adv = jnp.where(lt8, _shuf(u0, rot8), u1)
      w = jnp.exp(_leaky(asv + adv) - mv)
      plsc.store_scatter(wrows, [2 * i + half, colw], w)
      off = 4 * i + qoff
      plsc.store_scatter(wt, [hpvec, off >> 4, off & 15], w)

    for c in range(CH):
      pltpu.sync_copy(wrows.at[pl.ds(c * 128, 128)],
                      denacc.at[didx.at[c]], add=True)
    gbase = wid * (PER_W // 8) + blk * (B // 8)
    for hp in range(4):
      pltpu.sync_copy(wt.at[hp], wt_hbm.at[hp, pl.ds(gbase, B // 8)])

  plsc.subcore_barrier()
  pltpu.sync_copy(denacc.at[pl.ds(sid * ROWS_PT, ROWS_PT)],
                  den_hbm.at[cid, pl.ds(sid * ROWS_PT, ROWS_PT)])


def _make_pass_b(hp):
  @functools.partial(
      pl.kernel,
      out_type=jax.ShapeDtypeStruct((NC, NP1, 16), f32),
      mesh=_mesh,
      compiler_params=_sc_params,
      scratch_types=[
          pltpu.VMEM((CH2, 128), i32),
          pltpu.VMEM((CH2, 128), i32),
          pltpu.VMEM((CH2, 128), i32),
          pltpu.VMEM((B2, 16), f32),
          pltpu.VMEM((B2 // 8, 16), f32),
          pltpu.VMEM((B2, 16), f32),
          pltpu.VMEM_SHARED((NP1, 16), f32),
      ],
  )
  def _pass_b(ht_hbm, src_hbm, dst_hbm, w_hbm, z16_hbm, out_hbm,
              sidx, didx, idx4, gh, wv, msg, acc):
    cid = lax.axis_index("c")
    sid = lax.axis_index("s")
    wid = sid * NC + cid
    pltpu.sync_copy(z16_hbm, acc.at[pl.ds(sid * ROWS_PT, ROWS_PT)])
    plsc.subcore_barrier()
    iota = lax.iota(i32, 16)
    basepat = iota >> 3
    wrow_base = wid * (PER_W // 128)

    @pl.loop(0, NBLK2)
    def _blk(blk):
      rowb = wrow_base + blk * CH2
      pltpu.sync_copy(src_hbm.at[pl.ds(rowb, CH2)], sidx)
      pltpu.sync_copy(dst_hbm.at[pl.ds(rowb, CH2)], didx)

      @pl.loop(0, CH2)
      def _cvt(r):
        @pl.loop(0, 8)
        def _cvt16(k):
          idx4[r, pl.ds(k * 16, 16)] = sidx[r, pl.ds(k * 16, 16)] * 4 + hp

      for c in range(CH2):
        pltpu.sync_copy(ht_hbm.at[idx4.at[c]], gh.at[pl.ds(c * 128, 128)])
      gbase = wid * (PER_W // 8) + blk * (B2 // 8)
      pltpu.sync_copy(w_hbm.at[hp, pl.ds(gbase, B2 // 8)], wv)

      @pl.loop(0, B2 // 8)
      def _grp(g):
        wvec = wv[g]
        for t in range(8):
          m = _shuf(wvec, basepat + 2 * t)
          e = g * 8 + t
          msg[e] = gh[e] * m

      for c in range(CH2):
        pltpu.sync_copy(msg.at[pl.ds(c * 128, 128)],
                        acc.at[didx.at[c]], add=True)

    plsc.subcore_barrier()
    pltpu.sync_copy(acc.at[pl.ds(sid * ROWS_PT, ROWS_PT)],
                    out_hbm.at[cid, pl.ds(sid * ROWS_PT, ROWS_PT)])

  return _pass_b


_pass_b_fns = [_make_pass_b(hp) for hp in range(4)]


def _fin1_body(da_ref, db_ref, p00, p01, p02, p03, p10, p11, p12, p13,
               rep_ref, w2_ref, as2_ref, ad2_ref, b1_ref,
               t2_ref, bm2_ref):
  den = da_ref[...] + db_ref[...] + 1e-16
  msum = jnp.concatenate(
      [p00[...] + p10[...], p01[...] + p11[...],
       p02[...] + p12[...], p03[...] + p13[...]], axis=1)
  drep = jnp.dot(den, rep_ref[...], preferred_element_type=f32)
  out1 = msum / drep + b1_ref[...]
  hmid = jnp.where(out1 > 0, out1, jnp.exp(out1) - 1.0)
  h2 = jnp.dot(hmid, w2_ref[...], preferred_element_type=f32)
  a2s = jnp.sum(h2 * as2_ref[...], axis=1, keepdims=True)
  a2d = jnp.sum(h2 * ad2_ref[...], axis=1, keepdims=True)
  t2_ref[...] = jnp.concatenate(
      [h2, a2s, a2d, jnp.zeros((RB, 6), f32)], axis=1)
  bm2_ref[...] = jnp.concatenate(
      [jnp.full((1, 8, 64), jnp.max(a2s), f32),
       jnp.full((1, 8, 64), jnp.max(a2d), f32)], axis=2)


def _fin1(den, parts, rep, w2, as2, ad2, b1row):
  def full(shape):
    return pl.BlockSpec(shape, lambda i: tuple(0 for _ in shape))
  return pl.pallas_call(
      _fin1_body,
      grid=(GRID,),
      in_specs=[
          pl.BlockSpec((RB, 8), lambda i: (i, 0)),
          pl.BlockSpec((RB, 8), lambda i: (i, 0)),
      ] + [pl.BlockSpec((RB, 16), lambda i: (i, 0))] * 8 + [
          full((8, 64)), full((64, 8)), full((1, 8)), full((1, 8)),
          full((1, 64)),
      ],
      out_specs=[
          pl.BlockSpec((RB, 16), lambda i: (i, 0)),
          pl.BlockSpec((1, 8, 128), lambda i: (i, 0, 0)),
      ],
      out_shape=[
          jax.ShapeDtypeStruct((N, 16), f32),
          jax.ShapeDtypeStruct((GRID, 8, 128), f32),
      ],
  )(den[0], den[1], *parts, rep, w2, as2, ad2, b1row)


@functools.partial(
    pl.kernel,
    out_type=jax.ShapeDtypeStruct((NC, NP1, 16), f32),
    mesh=_mesh,
    compiler_params=_sc_params,
    scratch_types=[
        pltpu.VMEM((CH2, 128), i32),
        pltpu.VMEM((CH2, 128), i32),
        pltpu.VMEM((B2, 16), f32),
        pltpu.VMEM((B2, 16), f32),
        pltpu.VMEM((B2, 16), f32),
        pltpu.VMEM((16,), f32),
        pltpu.VMEM_SHARED((NP1, 16), f32),
    ],
)
def _pass_c(t2_hbm, src_hbm, dst_hbm, m_hbm, z16_hbm, out_hbm,
            sidx, didx, gs, gd, msg, mvec, acc):
  cid = lax.axis_index("c")
  sid = lax.axis_index("s")
  wid = sid * NC + cid
  pltpu.sync_copy(z16_hbm, acc.at[pl.ds(sid * ROWS_PT, ROWS_PT)])
  pltpu.sync_copy(m_hbm, mvec)
  plsc.subcore_barrier()
  mv = mvec[...]
  iota = lax.iota(i32, 16)
  rot1 = (iota + 1) & 15
  pat8 = (iota & 0) + 8
  c0 = jnp.where(iota < 8, 1.0, 0.0).astype(f32)
  c1 = jnp.where(iota == 8, 1.0, 0.0).astype(f32)
  wrow_base = wid * (PER_W // 128)

  @pl.loop(0, NBLK2)
  def _blk(blk):
    rowb = wrow_base + blk * CH2
    pltpu.sync_copy(src_hbm.at[pl.ds(rowb, CH2)], sidx)
    pltpu.sync_copy(dst_hbm.at[pl.ds(rowb, CH2)], didx)
    for c in range(CH2):
      pltpu.sync_copy(t2_hbm.at[sidx.at[c]], gs.at[pl.ds(c * 128, 128)])
      pltpu.sync_copy(t2_hbm.at[didx.at[c]], gd.at[pl.ds(c * 128, 128)])

    @pl.loop(0, B2)
    def _edge(i):
      v_s = gs[i]
      v_d = gd[i]
      t = v_s + _shuf(v_d, rot1)
      w = jnp.exp(_leaky(t) - mv)
      b = _shuf(w, pat8)
      msg[i] = (v_s * c0 + c1) * b

    for c in range(CH2):
      pltpu.sync_copy(msg.at[pl.ds(c * 128, 128)],
                      acc.at[didx.at[c]], add=True)

  plsc.subcore_barrier()
  pltpu.sync_copy(acc.at[pl.ds(sid * ROWS_PT, ROWS_PT)],
                  out_hbm.at[cid, pl.ds(sid * ROWS_PT, ROWS_PT)])


def _fin2_body(aa_ref, ab_ref, b2_ref, out_ref):
  s = aa_ref[...] + ab_ref[...]
  out_ref[...] = s[:, :8] / (s[:, 8:9] + 1e-16) + b2_ref[...]


def _fin2(acc2, b2row):
  return pl.pallas_call(
      _fin2_body,
      grid=(GRID,),
      in_specs=[
          pl.BlockSpec((RB, 16), lambda i: (i, 0)),
          pl.BlockSpec((RB, 16), lambda i: (i, 0)),
          pl.BlockSpec((1, 8), lambda i: (0, 0)),
      ],
      out_specs=pl.BlockSpec((RB, 8), lambda i: (i, 0)),
      out_shape=jax.ShapeDtypeStruct((N, 8), f32),
  )(acc2[0], acc2[1], b2row)


def kernel(x, edge_index, W1, a_src1, a_dst1, b1, W2, a_src2, a_dst2, b2):
  loops = jnp.arange(N, dtype=i32)
  src = jnp.concatenate([edge_index[0], loops, jnp.zeros((PAD,), i32)])
  dst = jnp.concatenate([edge_index[1], loops, jnp.full((PAD,), N, i32)])
  src2d = src.reshape(ETP // 128, 128)
  dst2d = dst.reshape(ETP // 128, 128)

  eye8 = jnp.eye(8, dtype=f32)
  pas = (eye8[:, None, :] * a_src1[:, :, None]).reshape(64, 8)
  padm = (eye8[:, None, :] * a_dst1[:, :, None]).reshape(64, 8)
  rep = jnp.repeat(eye8, 8, axis=1)

  z8 = jnp.zeros((ROWS_PT, 8), f32)
  z16 = jnp.zeros((ROWS_PT, 16), f32)

  h1, t1, bm1 = _tc1(x, W1, pas, padm)
  m1 = jnp.max(bm1[:, 0, 0]) + jnp.max(bm1[:, 0, 64])
  m1v = jnp.full((16,), m1, f32)
  t1p = jnp.concatenate([t1, jnp.zeros((1, 16), f32)], axis=0)
  ht = jnp.concatenate([h1, jnp.zeros((1, 64), f32)], axis=0)
  ht = ht.reshape((N + 1) * 4, 16)

  wt, den = _pass_a(t1p, src2d, dst2d, m1v, z8)
  bparts = [fn(ht, src2d, dst2d, wt, z16) for fn in _pass_b_fns]
  parts = [p[c][:N] for c in range(NC) for p in bparts]

  t2, bm2 = _fin1((den[0][:N], den[1][:N]), parts, rep, W2,
                  a_src2.reshape(1, 8), a_dst2.reshape(1, 8),
                  b1.reshape(1, 64))
  m2 = jnp.max(bm2[:, 0, 0]) + jnp.max(bm2[:, 0, 64])
  m2v = jnp.full((16,), m2, f32)
  t2p = jnp.concatenate([t2, jnp.zeros((1, 16), f32)], axis=0)

  acc2 = _pass_c(t2p, src2d, dst2d, m2v, z16)

  return _fin2((acc2[0][:N], acc2[1][:N]), b2.reshape(1, 8))


# speedup vs baseline: 62.5418x; 1.0730x over previous
import dataclasses
import functools

import jax
import jax.numpy as jnp
from jax import lax
from jax.experimental import pallas as pl
from jax.experimental.pallas import tpu as pltpu
from jax.experimental.pallas import tpu_sc as plsc

N = 100000
E = 3200000
NUM_FEAT = 16
DIM = 8
H1 = 8

ET = E + N
NC, NS = 2, 16
NW = NC * NS
B = 1024
CH = B // 128
NBLK = -(-ET // (NW * B))
ETP = NW * B * NBLK
B2 = 512
CH2 = B2 // 128
NBLK2 = ETP // (NW * B2)
PER_W = ETP // NW
PAD = ETP - ET
NP1 = 100096
ROWS_PT = NP1 // NS
RB = 1000
GRID = N // RB

_mesh = plsc.VectorSubcoreMesh(
    core_axis_name="c", subcore_axis_name="s", num_cores=NC, num_subcores=NS
)

f32 = jnp.float32
i32 = jnp.int32

_sc_params = pltpu.CompilerParams(use_tc_tiling_on_sc=False)
if "needs_layout_passes" in pltpu.CompilerParams.__dataclass_fields__:
  _sc_params = dataclasses.replace(_sc_params, needs_layout_passes=False)


def _leaky(x):
  return jnp.where(x >= 0, x, 0.2 * x)


def _shuf(v, idx):
  return jnp.take_along_axis(v, idx, axis=0, mode="promise_in_bounds")


def _tc1_body(x_ref, w1_ref, pas_ref, pad_ref, h_ref, t1_ref, bm_ref):
  h = jnp.dot(x_ref[...], w1_ref[...], preferred_element_type=f32)
  h_ref[...] = h
  a_s = jnp.dot(h, pas_ref[...], preferred_element_type=f32)
  a_d = jnp.dot(h, pad_ref[...], preferred_element_type=f32)
  t1_ref[...] = jnp.concatenate([a_s, a_d], axis=1)
  bm_ref[...] = jnp.concatenate(
      [jnp.full((1, 8, 64), jnp.max(a_s), f32),
       jnp.full((1, 8, 64), jnp.max(a_d), f32)], axis=2)


def _tc1(x, w1, pas, padm):
  return pl.pallas_call(
      _tc1_body,
      grid=(GRID,),
      in_specs=[
          pl.BlockSpec((RB, NUM_FEAT), lambda i: (i, 0)),
          pl.BlockSpec((NUM_FEAT, 64), lambda i: (0, 0)),
          pl.BlockSpec((64, 8), lambda i: (0, 0)),
          pl.BlockSpec((64, 8), lambda i: (0, 0)),
      ],
      out_specs=[
          pl.BlockSpec((RB, 64), lambda i: (i, 0)),
          pl.BlockSpec((RB, 16), lambda i: (i, 0)),
          pl.BlockSpec((1, 8, 128), lambda i: (i, 0, 0)),
      ],
      out_shape=[
          jax.ShapeDtypeStruct((N, 64), f32),
          jax.ShapeDtypeStruct((N, 16), f32),
          jax.ShapeDtypeStruct((GRID, 8, 128), f32),
      ],
  )(x, w1, pas, padm)


@functools.partial(
    pl.kernel,
    out_type=(
        jax.ShapeDtypeStruct((4, ETP // 8, 16), f32),
        jax.ShapeDtypeStruct((NC, NP1, 8), f32),
    ),
    mesh=_mesh,
    compiler_params=_sc_params,
    scratch_types=[
        pltpu.VMEM((CH, 128), i32),
        pltpu.VMEM((CH, 128), i32),
        pltpu.VMEM((B, 16), f32),
        pltpu.VMEM((B, 16), f32),
        pltpu.VMEM((B, 8), f32),
        pltpu.VMEM((4, B // 8, 16), f32),
        pltpu.VMEM((16,), f32),
        pltpu.VMEM_SHARED((NP1, 8), f32),
        pltpu.SemaphoreType.DMA,
    ],
)
def _pass_a(t1_hbm, src_hbm, dst_hbm, m_hbm, z8_hbm,
            wt_hbm, den_hbm,
            sidx, didx, gs, gd, wrows, wt, mvec, denacc, gsem):
  cid = lax.axis_index("c")
  sid = lax.axis_index("s")
  wid = sid * NC + cid
  pltpu.sync_copy(z8_hbm, denacc.at[pl.ds(sid * ROWS_PT, ROWS_PT)])
  pltpu.sync_copy(m_hbm, mvec)
  plsc.subcore_barrier()
  mv = mvec[...]
  iota = lax.iota(i32, 16)
  rot8 = iota ^ 8
  lt8 = iota < 8
  colw = iota & 7
  half = iota >> 3
  hpvec = (iota & 7) >> 1
  qoff = 2 * half + (iota & 1)
  wrow_base = wid * (PER_W // 128)

  @pl.loop(0, NBLK)
  def _blk(blk):
    rowb = wrow_base + blk * CH
    pltpu.sync_copy(src_hbm.at[pl.ds(rowb, CH)], sidx)
    pltpu.sync_copy(dst_hbm.at[pl.ds(rowb, CH)], didx)
    descs = []
    for c in range(CH):
      descs.append(pltpu.async_copy(
          t1_hbm.at[sidx.at[c]], gs.at[pl.ds(c * 128, 128)], gsem))
      descs.append(pltpu.async_copy(
          t1_hbm.at[didx.at[c]], gd.at[pl.ds(c * 128, 128)], gsem))
    for d in descs:
      d.wait()

    @pl.loop(0, B // 2, unroll=4)
    def _pair(i):
      v0 = gs[2 * i]
      v1 = gs[2 * i + 1]
      u0 = gd[2 * i]
      u1 = gd[2 * i + 1]
      asv = jnp.where(lt8, v0, _shuf(v1, rot8))
      adv = jnp.where(lt8, _shuf(u0, rot8), u1)
      w = jnp.exp(_leaky(asv + adv) - mv)
      plsc.store_scatter(wrows, [2 * i + half, colw], w)
      off = 4 * i + qoff
      plsc.store_scatter(wt, [hpvec, off >> 4, off & 15], w)

    for c in range(CH):
      pltpu.sync_copy(wrows.at[pl.ds(c * 128, 128)],
                      denacc.at[didx.at[c]], add=True)
    gbase = wid * (PER_W // 8) + blk * (B // 8)
    for hp in range(4):
      pltpu.sync_copy(wt.at[hp], wt_hbm.at[hp, pl.ds(gbase, B // 8)])

  plsc.subcore_barrier()
  pltpu.sync_copy(denacc.at[pl.ds(sid * ROWS_PT, ROWS_PT)],
                  den_hbm.at[cid, pl.ds(sid * ROWS_PT, ROWS_PT)])


def _make_pass_b(hp):
  @functools.partial(
      pl.kernel,
      out_type=jax.ShapeDtypeStruct((NC, NP1, 16), f32),
      mesh=_mesh,
      compiler_params=_sc_params,
      scratch_types=[
          pltpu.VMEM((CH2, 128), i32),
          pltpu.VMEM((CH2, 128), i32),
          pltpu.VMEM((CH2, 128), i32),
          pltpu.VMEM((B2, 16), f32),
          pltpu.VMEM((B2 // 8, 16), f32),
          pltpu.VMEM((B2, 16), f32),
          pltpu.VMEM_SHARED((NP1, 16), f32),
          pltpu.SemaphoreType.DMA,
      ],
  )
  def _pass_b(ht_hbm, src_hbm, dst_hbm, w_hbm, z16_hbm, out_hbm,
              sidx, didx, idx4, gh, wv, msg, acc, gsem):
    cid = lax.axis_index("c")
    sid = lax.axis_index("s")
    wid = sid * NC + cid
    pltpu.sync_copy(z16_hbm, acc.at[pl.ds(sid * ROWS_PT, ROWS_PT)])
    plsc.subcore_barrier()
    iota = lax.iota(i32, 16)
    basepat = iota >> 3
    wrow_base = wid * (PER_W // 128)

    @pl.loop(0, NBLK2)
    def _blk(blk):
      rowb = wrow_base + blk * CH2
      pltpu.sync_copy(src_hbm.at[pl.ds(rowb, CH2)], sidx)
      pltpu.sync_copy(dst_hbm.at[pl.ds(rowb, CH2)], didx)

      @pl.loop(0, CH2)
      def _cvt(r):
        @pl.loop(0, 8)
        def _cvt16(k):
          idx4[r, pl.ds(k * 16, 16)] = sidx[r, pl.ds(k * 16, 16)] * 4 + hp

      descs = []
      for c in range(CH2):
        descs.append(pltpu.async_copy(
            ht_hbm.at[idx4.at[c]], gh.at[pl.ds(c * 128, 128)], gsem))
      gbase = wid * (PER_W // 8) + blk * (B2 // 8)
      descs.append(pltpu.async_copy(
          w_hbm.at[hp, pl.ds(gbase, B2 // 8)], wv, gsem))
      for d in descs:
        d.wait()

      @pl.loop(0, B2 // 8, unroll=2)
      def _grp(g):
        wvec = wv[g]
        for t in range(8):
          m = _shuf(wvec, basepat + 2 * t)
          e = g * 8 + t
          msg[e] = gh[e] * m

      for c in range(CH2):
        pltpu.sync_copy(msg.at[pl.ds(c * 128, 128)],
                        acc.at[didx.at[c]], add=True)

    plsc.subcore_barrier()
    pltpu.sync_copy(acc.at[pl.ds(sid * ROWS_PT, ROWS_PT)],
                    out_hbm.at[cid, pl.ds(sid * ROWS_PT, ROWS_PT)])

  return _pass_b


_pass_b_fns = [_make_pass_b(hp) for hp in range(4)]


def _fin1_body(da_ref, db_ref, p00, p01, p02, p03, p10, p11, p12, p13,
               rep_ref, w2_ref, as2_ref, ad2_ref, b1_ref,
               t2_ref, bm2_ref):
  den = da_ref[...] + db_ref[...] + 1e-16
  msum = jnp.concatenate(
      [p00[...] + p10[...], p01[...] + p11[...],
       p02[...] + p12[...], p03[...] + p13[...]], axis=1)
  drep = jnp.dot(den, rep_ref[...], preferred_element_type=f32)
  out1 = msum / drep + b1_ref[...]
  hmid = jnp.where(out1 > 0, out1, jnp.exp(out1) - 1.0)
  h2 = jnp.dot(hmid, w2_ref[...], preferred_element_type=f32)
  a2s = jnp.sum(h2 * as2_ref[...], axis=1, keepdims=True)
  a2d = jnp.sum(h2 * ad2_ref[...], axis=1, keepdims=True)
  t2_ref[...] = jnp.concatenate(
      [h2, a2s, a2d, jnp.zeros((RB, 6), f32)], axis=1)
  bm2_ref[...] = jnp.concatenate(
      [jnp.full((1, 8, 64), jnp.max(a2s), f32),
       jnp.full((1, 8, 64), jnp.max(a2d), f32)], axis=2)


def _fin1(den, parts, rep, w2, as2, ad2, b1row):
  def full(shape):
    return pl.BlockSpec(shape, lambda i: tuple(0 for _ in shape))
  return pl.pallas_call(
      _fin1_body,
      grid=(GRID,),
      in_specs=[
          pl.BlockSpec((RB, 8), lambda i: (i, 0)),
          pl.BlockSpec((RB, 8), lambda i: (i, 0)),
      ] + [pl.BlockSpec((RB, 16), lambda i: (i, 0))] * 8 + [
          full((8, 64)), full((64, 8)), full((1, 8)), full((1, 8)),
          full((1, 64)),
      ],
      out_specs=[
          pl.BlockSpec((RB, 16), lambda i: (i, 0)),
          pl.BlockSpec((1, 8, 128), lambda i: (i, 0, 0)),
      ],
      out_shape=[
          jax.ShapeDtypeStruct((N, 16), f32),
          jax.ShapeDtypeStruct((GRID, 8, 128), f32),
      ],
  )(den[0], den[1], *parts, rep, w2, as2, ad2, b1row)


@functools.partial(
    pl.kernel,
    out_type=jax.ShapeDtypeStruct((NC, NP1, 16), f32),
    mesh=_mesh,
    compiler_params=_sc_params,
    scratch_types=[
        pltpu.VMEM((CH2, 128), i32),
        pltpu.VMEM((CH2, 128), i32),
        pltpu.VMEM((B2, 16), f32),
        pltpu.VMEM((B2, 16), f32),
        pltpu.VMEM((B2, 16), f32),
        pltpu.VMEM((16,), f32),
        pltpu.VMEM_SHARED((NP1, 16), f32),
        pltpu.SemaphoreType.DMA,
    ],
)
def _pass_c(t2_hbm, src_hbm, dst_hbm, m_hbm, z16_hbm, out_hbm,
            sidx, didx, gs, gd, msg, mvec, acc, gsem):
  cid = lax.axis_index("c")
  sid = lax.axis_index("s")
  wid = sid * NC + cid
  pltpu.sync_copy(z16_hbm, acc.at[pl.ds(sid * ROWS_PT, ROWS_PT)])
  pltpu.sync_copy(m_hbm, mvec)
  plsc.subcore_barrier()
  mv = mvec[...]
  iota = lax.iota(i32, 16)
  rot1 = (iota + 1) & 15
  pat8 = (iota & 0) + 8
  c0 = jnp.where(iota < 8, 1.0, 0.0).astype(f32)
  c1 = jnp.where(iota == 8, 1.0, 0.0).astype(f32)
  wrow_base = wid * (PER_W // 128)

  @pl.loop(0, NBLK2)
  def _blk(blk):
    rowb = wrow_base + blk * CH2
    pltpu.sync_copy(src_hbm.at[pl.ds(rowb, CH2)], sidx)
    pltpu.sync_copy(dst_hbm.at[pl.ds(rowb, CH2)], didx)
    descs = []
    for c in range(CH2):
      descs.append(pltpu.async_copy(
          t2_hbm.at[sidx.at[c]], gs.at[pl.ds(c * 128, 128)], gsem))
      descs.append(pltpu.async_copy(
          t2_hbm.at[didx.at[c]], gd.at[pl.ds(c * 128, 128)], gsem))
    for d in descs:
      d.wait()

    @pl.loop(0, B2, unroll=4)
    def _edge(i):
      v_s = gs[i]
      v_d = gd[i]
      t = v_s + _shuf(v_d, rot1)
      w = jnp.exp(_leaky(t) - mv)
      b = _shuf(w, pat8)
      msg[i] = (v_s * c0 + c1) * b

    for c in range(CH2):
      pltpu.sync_copy(msg.at[pl.ds(c * 128, 128)],
                      acc.at[didx.at[c]], add=True)

  plsc.subcore_barrier()
  pltpu.sync_copy(acc.at[pl.ds(sid * ROWS_PT, ROWS_PT)],
                  out_hbm.at[cid, pl.ds(sid * ROWS_PT, ROWS_PT)])


def _fin2_body(aa_ref, ab_ref, b2_ref, out_ref):
  s = aa_ref[...] + ab_ref[...]
  out_ref[...] = s[:, :8] / (s[:, 8:9] + 1e-16) + b2_ref[...]


def _fin2(acc2, b2row):
  return pl.pallas_call(
      _fin2_body,
      grid=(GRID,),
      in_specs=[
          pl.BlockSpec((RB, 16), lambda i: (i, 0)),
          pl.BlockSpec((RB, 16), lambda i: (i, 0)),
          pl.BlockSpec((1, 8), lambda i: (0, 0)),
      ],
      out_specs=pl.BlockSpec((RB, 8), lambda i: (i, 0)),
      out_shape=jax.ShapeDtypeStruct((N, 8), f32),
  )(acc2[0], acc2[1], b2row)


def kernel(x, edge_index, W1, a_src1, a_dst1, b1, W2, a_src2, a_dst2, b2):
  loops = jnp.arange(N, dtype=i32)
  src = jnp.concatenate([edge_index[0], loops, jnp.zeros((PAD,), i32)])
  dst = jnp.concatenate([edge_index[1], loops, jnp.full((PAD,), N, i32)])
  src2d = src.reshape(ETP // 128, 128)
  dst2d = dst.reshape(ETP // 128, 128)

  eye8 = jnp.eye(8, dtype=f32)
  pas = (eye8[:, None, :] * a_src1[:, :, None]).reshape(64, 8)
  padm = (eye8[:, None, :] * a_dst1[:, :, None]).reshape(64, 8)
  rep = jnp.repeat(eye8, 8, axis=1)

  z8 = jnp.zeros((ROWS_PT, 8), f32)
  z16 = jnp.zeros((ROWS_PT, 16), f32)

  h1, t1, bm1 = _tc1(x, W1, pas, padm)
  m1 = jnp.max(bm1[:, 0, 0]) + jnp.max(bm1[:, 0, 64])
  m1v = jnp.full((16,), m1, f32)
  t1p = jnp.concatenate([t1, jnp.zeros((1, 16), f32)], axis=0)
  ht = jnp.concatenate([h1, jnp.zeros((1, 64), f32)], axis=0)
  ht = ht.reshape((N + 1) * 4, 16)

  wt, den = _pass_a(t1p, src2d, dst2d, m1v, z8)
  bparts = [fn(ht, src2d, dst2d, wt, z16) for fn in _pass_b_fns]
  parts = [p[c][:N] for c in range(NC) for p in bparts]

  t2, bm2 = _fin1((den[0][:N], den[1][:N]), parts, rep, W2,
                  a_src2.reshape(1, 8), a_dst2.reshape(1, 8),
                  b1.reshape(1, 64))
  m2 = jnp.max(bm2[:, 0, 0]) + jnp.max(bm2[:, 0, 64])
  m2v = jnp.full((16,), m2, f32)
  t2p = jnp.concatenate([t2, jnp.zeros((1, 16), f32)], axis=0)

  acc2 = _pass_c(t2p, src2d, dst2d, m2v, z16)

  return _fin2((acc2[0][:N], acc2[1][:N]), b2.reshape(1, 8))


# speedup vs baseline: 86.8773x; 1.3891x over previous
import dataclasses
import functools

import jax
import jax.numpy as jnp
from jax import lax
from jax.experimental import pallas as pl
from jax.experimental.pallas import tpu as pltpu
from jax.experimental.pallas import tpu_sc as plsc

N = 100000
E = 3200000
NUM_FEAT = 16
DIM = 8
H1 = 8

ET = E + N
NC, NS = 2, 16
NW = NC * NS
B = 1024
CH = B // 128
NBLK = -(-ET // (NW * B))
ETP = NW * B * NBLK
B2 = 512
CH2 = B2 // 128
NBLK2 = ETP // (NW * B2)
PER_W = ETP // NW
PAD = ETP - ET
NP1 = 100096
ROWS_PT = NP1 // NS
RB = 1000
GRID = N // RB

_mesh = plsc.VectorSubcoreMesh(
    core_axis_name="c", subcore_axis_name="s", num_cores=NC, num_subcores=NS
)

f32 = jnp.float32
i32 = jnp.int32

_sc_params = pltpu.CompilerParams(use_tc_tiling_on_sc=False)
if "needs_layout_passes" in pltpu.CompilerParams.__dataclass_fields__:
  _sc_params = dataclasses.replace(_sc_params, needs_layout_passes=False)


def _leaky(x):
  return jnp.where(x >= 0, x, 0.2 * x)


def _shuf(v, idx):
  return jnp.take_along_axis(v, idx, axis=0, mode="promise_in_bounds")


def _tc1_body(x_ref, w1_ref, pas_ref, pad_ref, h_ref, t1_ref, bm_ref):
  h = jnp.dot(x_ref[...], w1_ref[...], preferred_element_type=f32)
  h_ref[...] = h
  a_s = jnp.dot(h, pas_ref[...], preferred_element_type=f32)
  a_d = jnp.dot(h, pad_ref[...], preferred_element_type=f32)
  t1_ref[...] = jnp.concatenate([a_s, a_d], axis=1)
  bm_ref[...] = jnp.concatenate(
      [jnp.full((1, 8, 64), jnp.max(a_s), f32),
       jnp.full((1, 8, 64), jnp.max(a_d), f32)], axis=2)


def _tc1(x, w1, pas, padm):
  return pl.pallas_call(
      _tc1_body,
      grid=(GRID,),
      in_specs=[
          pl.BlockSpec((RB, NUM_FEAT), lambda i: (i, 0)),
          pl.BlockSpec((NUM_FEAT, 64), lambda i: (0, 0)),
          pl.BlockSpec((64, 8), lambda i: (0, 0)),
          pl.BlockSpec((64, 8), lambda i: (0, 0)),
      ],
      out_specs=[
          pl.BlockSpec((RB, 64), lambda i: (i, 0)),
          pl.BlockSpec((RB, 16), lambda i: (i, 0)),
          pl.BlockSpec((1, 8, 128), lambda i: (i, 0, 0)),
      ],
      out_shape=[
          jax.ShapeDtypeStruct((N, 64), f32),
          jax.ShapeDtypeStruct((N, 16), f32),
          jax.ShapeDtypeStruct((GRID, 8, 128), f32),
      ],
  )(x, w1, pas, padm)


@functools.partial(
    pl.kernel,
    out_type=(
        jax.ShapeDtypeStruct((4, ETP // 8, 16), f32),
        jax.ShapeDtypeStruct((NC, NP1, 8), f32),
    ),
    mesh=_mesh,
    compiler_params=_sc_params,
    scratch_types=[
        pltpu.VMEM((CH, 128), i32),
        pltpu.VMEM((CH, 128), i32),
        pltpu.VMEM((B, 16), f32),
        pltpu.VMEM((B, 16), f32),
        pltpu.VMEM((B, 8), f32),
        pltpu.VMEM((4, B // 8, 16), f32),
        pltpu.VMEM((16,), f32),
        pltpu.VMEM_SHARED((NP1, 8), f32),
        pltpu.SemaphoreType.DMA,
    ],
)
def _pass_a(t1_hbm, src_hbm, dst_hbm, m_hbm, z8_hbm,
            wt_hbm, den_hbm,
            sidx, didx, gs, gd, wrows, wt, mvec, denacc, gsem):
  cid = lax.axis_index("c")
  sid = lax.axis_index("s")
  wid = sid * NC + cid
  pltpu.sync_copy(z8_hbm, denacc.at[pl.ds(sid * ROWS_PT, ROWS_PT)])
  pltpu.sync_copy(m_hbm, mvec)
  plsc.subcore_barrier()
  mv = mvec[...]
  iota = lax.iota(i32, 16)
  rot8 = iota ^ 8
  lt8 = iota < 8
  colw = iota & 7
  half = iota >> 3
  hpvec = (iota & 7) >> 1
  qoff = 2 * half + (iota & 1)
  wrow_base = wid * (PER_W // 128)

  @pl.loop(0, NBLK)
  def _blk(blk):
    rowb = wrow_base + blk * CH
    pltpu.sync_copy(src_hbm.at[pl.ds(rowb, CH)], sidx)
    pltpu.sync_copy(dst_hbm.at[pl.ds(rowb, CH)], didx)
    descs = []
    for c in range(CH):
      descs.append(pltpu.async_copy(
          t1_hbm.at[sidx.at[c]], gs.at[pl.ds(c * 128, 128)], gsem))
      descs.append(pltpu.async_copy(
          t1_hbm.at[didx.at[c]], gd.at[pl.ds(c * 128, 128)], gsem))
    for d in descs:
      d.wait()

    @pl.loop(0, B // 2, unroll=4)
    def _pair(i):
      v0 = gs[2 * i]
      v1 = gs[2 * i + 1]
      u0 = gd[2 * i]
      u1 = gd[2 * i + 1]
      asv = jnp.where(lt8, v0, _shuf(v1, rot8))
      adv = jnp.where(lt8, _shuf(u0, rot8), u1)
      w = jnp.exp(_leaky(asv + adv) - mv)
      plsc.store_scatter(wrows, [2 * i + half, colw], w)
      off = 4 * i + qoff
      plsc.store_scatter(wt, [hpvec, off >> 4, off & 15], w)

    for c in range(CH):
      pltpu.sync_copy(wrows.at[pl.ds(c * 128, 128)],
                      denacc.at[didx.at[c]], add=True)
    gbase = wid * (PER_W // 8) + blk * (B // 8)
    for hp in range(4):
      pltpu.sync_copy(wt.at[hp], wt_hbm.at[hp, pl.ds(gbase, B // 8)])

  plsc.subcore_barrier()
  pltpu.sync_copy(denacc.at[pl.ds(sid * ROWS_PT, ROWS_PT)],
                  den_hbm.at[cid, pl.ds(sid * ROWS_PT, ROWS_PT)])


@functools.partial(
    pl.kernel,
    out_type=jax.ShapeDtypeStruct((4, NC, NP1, 16), f32),
    mesh=_mesh,
    compiler_params=_sc_params,
    scratch_types=[
        pltpu.VMEM((CH2, 128), i32),
        pltpu.VMEM((CH2, 128), i32),
        pltpu.VMEM((CH2, 128), i32),
        pltpu.VMEM((B2, 16), f32),
        pltpu.VMEM((B2 // 8, 16), f32),
        pltpu.VMEM_SHARED((NP1, 16), f32),
        pltpu.SemaphoreType.DMA,
    ],
)
def _pass_b(ht_hbm, src_hbm, dst_hbm, w_hbm, z16_hbm, out_hbm,
            sidx, didx, idx4, gh, wv, acc, gsem):
  cid = lax.axis_index("c")
  sid = lax.axis_index("s")
  wid = sid * NC + cid
  iota = lax.iota(i32, 16)
  basepat = iota >> 3
  wrow_base = wid * (PER_W // 128)

  for hp in range(4):
    pltpu.sync_copy(z16_hbm, acc.at[pl.ds(sid * ROWS_PT, ROWS_PT)])
    plsc.subcore_barrier()

    @pl.loop(0, NBLK2)
    def _blk(blk):
      rowb = wrow_base + blk * CH2
      pltpu.sync_copy(src_hbm.at[pl.ds(rowb, CH2)], sidx)
      pltpu.sync_copy(dst_hbm.at[pl.ds(rowb, CH2)], didx)

      @pl.loop(0, CH2)
      def _cvt(r):
        @pl.loop(0, 8)
        def _cvt16(k):
          idx4[r, pl.ds(k * 16, 16)] = sidx[r, pl.ds(k * 16, 16)] * 4 + hp

      descs = []
      for c in range(CH2):
        descs.append(pltpu.async_copy(
            ht_hbm.at[idx4.at[c]], gh.at[pl.ds(c * 128, 128)], gsem))
      gbase = wid * (PER_W // 8) + blk * (B2 // 8)
      descs.append(pltpu.async_copy(
          w_hbm.at[hp, pl.ds(gbase, B2 // 8)], wv, gsem))
      for d in descs:
        d.wait()

      @pl.loop(0, B2 // 8, unroll=2)
      def _grp(g):
        wvec = wv[g]
        for t in range(8):
          m = _shuf(wvec, basepat + 2 * t)
          e = g * 8 + t
          gh[e] = gh[e] * m

      for c in range(CH2):
        pltpu.sync_copy(gh.at[pl.ds(c * 128, 128)],
                        acc.at[didx.at[c]], add=True)

    plsc.subcore_barrier()
    pltpu.sync_copy(acc.at[pl.ds(sid * ROWS_PT, ROWS_PT)],
                    out_hbm.at[hp, cid, pl.ds(sid * ROWS_PT, ROWS_PT)])
    plsc.subcore_barrier()


def _fin1_body(da_ref, db_ref, p00, p01, p02, p03, p10, p11, p12, p13,
               rep_ref, w2_ref, as2_ref, ad2_ref, b1_ref,
               t2_ref, as2o_ref, ad2o_ref, bm2_ref):
  den = da_ref[...] + db_ref[...] + 1e-16
  msum = jnp.concatenate(
      [p00[...] + p10[...], p01[...] + p11[...],
       p02[...] + p12[...], p03[...] + p13[...]], axis=1)
  drep = jnp.dot(den, rep_ref[...], preferred_element_type=f32)
  out1 = msum / drep + b1_ref[...]
  hmid = jnp.where(out1 > 0, out1, jnp.exp(out1) - 1.0)
  h2 = jnp.dot(hmid, w2_ref[...], preferred_element_type=f32)
  a2s = jnp.sum(h2 * as2_ref[...], axis=1, keepdims=True)
  a2d = jnp.sum(h2 * ad2_ref[...], axis=1, keepdims=True)
  t2_ref[...] = jnp.concatenate(
      [h2, jnp.ones((RB, 1), f32), jnp.zeros((RB, 7), f32)], axis=1)
  as2o_ref[...] = a2s
  ad2o_ref[...] = a2d
  bm2_ref[...] = jnp.concatenate(
      [jnp.full((1, 8, 64), jnp.max(a2s), f32),
       jnp.full((1, 8, 64), jnp.max(a2d), f32)], axis=2)


def _fin1(den, parts, rep, w2, as2, ad2, b1row):
  def full(shape):
    return pl.BlockSpec(shape, lambda i: tuple(0 for _ in shape))
  return pl.pallas_call(
      _fin1_body,
      grid=(GRID,),
      in_specs=[
          pl.BlockSpec((RB, 8), lambda i: (i, 0)),
          pl.BlockSpec((RB, 8), lambda i: (i, 0)),
      ] + [pl.BlockSpec((RB, 16), lambda i: (i, 0))] * 8 + [
          full((8, 64)), full((64, 8)), full((1, 8)), full((1, 8)),
          full((1, 64)),
      ],
      out_specs=[
          pl.BlockSpec((RB, 16), lambda i: (i, 0)),
          pl.BlockSpec((RB, 1), lambda i: (i, 0)),
          pl.BlockSpec((RB, 1), lambda i: (i, 0)),
          pl.BlockSpec((1, 8, 128), lambda i: (i, 0, 0)),
      ],
      out_shape=[
          jax.ShapeDtypeStruct((N, 16), f32),
          jax.ShapeDtypeStruct((N, 1), f32),
          jax.ShapeDtypeStruct((N, 1), f32),
          jax.ShapeDtypeStruct((GRID, 8, 128), f32),
      ],
  )(den[0], den[1], *parts, rep, w2, as2, ad2, b1row)


@functools.partial(
    pl.kernel,
    out_type=jax.ShapeDtypeStruct((NC, NP1, 16), f32),
    mesh=_mesh,
    compiler_params=_sc_params,
    scratch_types=[
        pltpu.VMEM((CH2, 128), i32),
        pltpu.VMEM((CH2, 128), i32),
        pltpu.VMEM((B2, 16), f32),
        pltpu.VMEM((B2,), f32),
        pltpu.VMEM((B2,), f32),
        pltpu.VMEM((16,), f32),
        pltpu.VMEM_SHARED((NP1, 16), f32),
        pltpu.SemaphoreType.DMA,
    ],
)
def _pass_c(t2_hbm, as2_hbm, ad2_hbm, src_hbm, dst_hbm, m_hbm, z16_hbm,
            out_hbm,
            sidx, didx, gs, asg, adg, mvec, acc, gsem):
  cid = lax.axis_index("c")
  sid = lax.axis_index("s")
  wid = sid * NC + cid
  pltpu.sync_copy(z16_hbm, acc.at[pl.ds(sid * ROWS_PT, ROWS_PT)])
  pltpu.sync_copy(m_hbm, mvec)
  plsc.subcore_barrier()
  mv = mvec[...]
  iota = lax.iota(i32, 16)
  bpats = [iota * 0 + t for t in range(16)]
  wrow_base = wid * (PER_W // 128)

  @pl.loop(0, NBLK2)
  def _blk(blk):
    rowb = wrow_base + blk * CH2
    pltpu.sync_copy(src_hbm.at[pl.ds(rowb, CH2)], sidx)
    pltpu.sync_copy(dst_hbm.at[pl.ds(rowb, CH2)], didx)
    descs = []
    for c in range(CH2):
      descs.append(pltpu.async_copy(
          t2_hbm.at[sidx.at[c]], gs.at[pl.ds(c * 128, 128)], gsem))
      descs.append(pltpu.async_copy(
          as2_hbm.at[sidx.at[c]], asg.at[pl.ds(c * 128, 128)], gsem))
      descs.append(pltpu.async_copy(
          ad2_hbm.at[didx.at[c]], adg.at[pl.ds(c * 128, 128)], gsem))
    for d in descs:
      d.wait()

    @pl.loop(0, B2 // 16)
    def _grp(j):
      ev = jnp.exp(
          _leaky(asg[pl.ds(j * 16, 16)] + adg[pl.ds(j * 16, 16)]) - mv)
      for t in range(16):
        e = j * 16 + t
        gs[e] = gs[e] * _shuf(ev, bpats[t])

    for c in range(CH2):
      pltpu.sync_copy(gs.at[pl.ds(c * 128, 128)],
                      acc.at[didx.at[c]], add=True)

  plsc.subcore_barrier()
  pltpu.sync_copy(acc.at[pl.ds(sid * ROWS_PT, ROWS_PT)],
                  out_hbm.at[cid, pl.ds(sid * ROWS_PT, ROWS_PT)])


def _fin2_body(aa_ref, ab_ref, b2_ref, out_ref):
  s = aa_ref[...] + ab_ref[...]
  out_ref[...] = s[:, :8] / (s[:, 8:9] + 1e-16) + b2_ref[...]


def _fin2(acc2, b2row):
  return pl.pallas_call(
      _fin2_body,
      grid=(GRID,),
      in_specs=[
          pl.BlockSpec((RB, 16), lambda i: (i, 0)),
          pl.BlockSpec((RB, 16), lambda i: (i, 0)),
          pl.BlockSpec((1, 8), lambda i: (0, 0)),
      ],
      out_specs=pl.BlockSpec((RB, 8), lambda i: (i, 0)),
      out_shape=jax.ShapeDtypeStruct((N, 8), f32),
  )(acc2[0], acc2[1], b2row)


def kernel(x, edge_index, W1, a_src1, a_dst1, b1, W2, a_src2, a_dst2, b2):
  loops = jnp.arange(N, dtype=i32)
  src = jnp.concatenate([edge_index[0], loops, jnp.zeros((PAD,), i32)])
  dst = jnp.concatenate([edge_index[1], loops, jnp.full((PAD,), N, i32)])
  src2d = src.reshape(ETP // 128, 128)
  dst2d = dst.reshape(ETP // 128, 128)

  eye8 = jnp.eye(8, dtype=f32)
  pas = (eye8[:, None, :] * a_src1[:, :, None]).reshape(64, 8)
  padm = (eye8[:, None, :] * a_dst1[:, :, None]).reshape(64, 8)
  rep = jnp.repeat(eye8, 8, axis=1)

  z8 = jnp.zeros((ROWS_PT, 8), f32)
  z16 = jnp.zeros((ROWS_PT, 16), f32)

  h1, t1, bm1 = _tc1(x, W1, pas, padm)
  m1 = jnp.max(bm1[:, 0, 0]) + jnp.max(bm1[:, 0, 64])
  m1v = jnp.full((16,), m1, f32)
  t1p = jnp.concatenate([t1, jnp.zeros((1, 16), f32)], axis=0)
  ht = jnp.concatenate([h1, jnp.zeros((1, 64), f32)], axis=0)
  ht = ht.reshape((N + 1) * 4, 16)

  wt, den = _pass_a(t1p, src2d, dst2d, m1v, z8)
  bparts = _pass_b(ht, src2d, dst2d, wt, z16)
  parts = [bparts[hp, c][:N] for c in range(NC) for hp in range(4)]

  t2, as2t, ad2t, bm2 = _fin1((den[0][:N], den[1][:N]), parts, rep, W2,
                              a_src2.reshape(1, 8), a_dst2.reshape(1, 8),
                              b1.reshape(1, 64))
  m2 = jnp.max(bm2[:, 0, 0]) + jnp.max(bm2[:, 0, 64])
  m2v = jnp.full((16,), m2, f32)
  t2p = jnp.concatenate([t2, jnp.zeros((1, 16), f32)], axis=0)
  as2p = jnp.concatenate([as2t[:, 0], jnp.zeros((1,), f32)])
  ad2p = jnp.concatenate([ad2t[:, 0], jnp.zeros((1,), f32)])

  acc2 = _pass_c(t2p, as2p, ad2p, src2d, dst2d, m2v, z16)

  return _fin2((acc2[0][:N], acc2[1][:N]), b2.reshape(1, 8))
